# Initial kernel scaffold; baseline (speedup 1.0000x reference)
import functools

import jax
import jax.numpy as jnp
from jax import lax
from jax.experimental import pallas as pl
from jax.experimental.pallas import tpu as pltpu
from jax.experimental.pallas import tpu_sc as plsc

N = 10000
D = 128
E = 320000
N_ACC = 10016
ROWS_PER_TILE = N_ACC // 16
C = 128
NCHUNK_A = 157
NCHUNK_C = 79
CNTW = 16

_MESH = plsc.VectorSubcoreMesh(core_axis_name="c", subcore_axis_name="s")


@functools.partial(
    pl.kernel,
    out_type=[
        jax.ShapeDtypeStruct((2, N_ACC, D), jnp.float32),
        jax.ShapeDtypeStruct((2, N_ACC, CNTW), jnp.float32),
    ],
    mesh=_MESH,
    scratch_types=[
        pltpu.VMEM((NCHUNK_A, C), jnp.int32),
        pltpu.VMEM((NCHUNK_A, C), jnp.int32),
        pltpu.VMEM((C, D), jnp.float32),
        pltpu.VMEM((C, CNTW), jnp.float32),
        pltpu.VMEM_SHARED((N_ACC, D), jnp.float32),
        pltpu.VMEM_SHARED((N_ACC, CNTW), jnp.float32),
        pltpu.SemaphoreType.DMA,
    ],
)
def _sc_stage_a(table, src_idx, dst_idx, ones16, zeros_f, zeros_c,
                sums, cnts, idx_s, idx_d, rows, ones_v, acc, cacc, sem):
    c = lax.axis_index("c")
    s = lax.axis_index("s")
    r0 = s * ROWS_PER_TILE
    pltpu.sync_copy(src_idx.at[c, s], idx_s)
    pltpu.sync_copy(dst_idx.at[c, s], idx_d)
    pltpu.sync_copy(ones16, ones_v)
    pltpu.sync_copy(zeros_f.at[pl.ds(r0, ROWS_PER_TILE)],
                    acc.at[pl.ds(r0, ROWS_PER_TILE)])
    pltpu.sync_copy(zeros_c.at[pl.ds(r0, ROWS_PER_TILE)],
                    cacc.at[pl.ds(r0, ROWS_PER_TILE)])
    plsc.subcore_barrier()

    def chunk(j, carry):
        pltpu.async_copy(table.at[idx_s.at[j]], rows, sem).wait()
        pltpu.sync_copy(rows, acc.at[idx_d.at[j]], add=True)
        pltpu.sync_copy(ones_v, cacc.at[idx_d.at[j]], add=True)
        return carry

    lax.fori_loop(0, NCHUNK_A, chunk, 0)
    plsc.subcore_barrier()
    pltpu.sync_copy(acc.at[pl.ds(r0, ROWS_PER_TILE)],
                    sums.at[c, pl.ds(r0, ROWS_PER_TILE)])
    pltpu.sync_copy(cacc.at[pl.ds(r0, ROWS_PER_TILE)],
                    cnts.at[c, pl.ds(r0, ROWS_PER_TILE)])


@functools.partial(
    pl.kernel,
    out_type=jax.ShapeDtypeStruct((2, N_ACC, D), jnp.float32),
    mesh=_MESH,
    scratch_types=[
        pltpu.VMEM((NCHUNK_C, C), jnp.int32),
        pltpu.VMEM((NCHUNK_C, C), jnp.int32),
        pltpu.VMEM((C, D), jnp.float32),
        pltpu.VMEM_SHARED((N_ACC, D), jnp.float32),
        pltpu.SemaphoreType.DMA,
    ],
)
def _sc_stage_c(table, src_idx, dst_idx, zeros_f,
                sums, idx_s, idx_d, rows, acc, sem):
    c = lax.axis_index("c")
    s = lax.axis_index("s")
    r0 = s * ROWS_PER_TILE
    pltpu.sync_copy(src_idx.at[c, s], idx_s)
    pltpu.sync_copy(dst_idx.at[c, s], idx_d)
    pltpu.sync_copy(zeros_f.at[pl.ds(r0, ROWS_PER_TILE)],
                    acc.at[pl.ds(r0, ROWS_PER_TILE)])
    plsc.subcore_barrier()

    def chunk(j, carry):
        pltpu.async_copy(table.at[idx_s.at[j]], rows, sem).wait()
        pltpu.sync_copy(rows, acc.at[idx_d.at[j]], add=True)
        return carry

    lax.fori_loop(0, NCHUNK_C, chunk, 0)
    plsc.subcore_barrier()
    pltpu.sync_copy(acc.at[pl.ds(r0, ROWS_PER_TILE)],
                    sums.at[c, pl.ds(r0, ROWS_PER_TILE)])


_BLK = 1000


def _tc_update_body(s_ref, c_ref, x_ref, w_ref, b_ref, o_ref):
    inv = 1.0 / jnp.maximum(c_ref[0, :, 0:1], 1.0)
    agg = s_ref[0] * inv
    xcat = jnp.concatenate([agg, x_ref[...]], axis=1)
    o_ref[...] = jnp.maximum(
        jnp.dot(xcat, w_ref[...], preferred_element_type=jnp.float32)
        + b_ref[...], 0.0)


def _tc_update(sums, cnts, rel, x, w, b):
    return pl.pallas_call(
        _tc_update_body,
        out_shape=jax.ShapeDtypeStruct((N, D), jnp.float32),
        grid=(N // _BLK,),
        in_specs=[
            pl.BlockSpec((1, _BLK, D), lambda i: (rel, i, 0)),
            pl.BlockSpec((1, _BLK, CNTW), lambda i: (rel, i, 0)),
            pl.BlockSpec((_BLK, D), lambda i: (i, 0)),
            pl.BlockSpec((2 * D, D), lambda i: (0, 0)),
            pl.BlockSpec((1, D), lambda i: (0, 0)),
        ],
        out_specs=pl.BlockSpec((_BLK, D), lambda i: (i, 0)),
    )(sums, cnts, x, w, b.reshape(1, D))


def _tc_update2_body(s_ref, c_ref, x_ref, w_ref, b_ref, o_ref):
    inv = 1.0 / jnp.maximum(c_ref[0, :, 0:1], 1.0)
    agg = (s_ref[0] + s_ref[1]) * inv
    xcat = jnp.concatenate([agg, x_ref[...]], axis=1)
    o_ref[...] = jnp.maximum(
        jnp.dot(xcat, w_ref[...], preferred_element_type=jnp.float32)
        + b_ref[...], 0.0)


def _tc_update2(sums2, cnts, rel, x, w, b):
    return pl.pallas_call(
        _tc_update2_body,
        out_shape=jax.ShapeDtypeStruct((N, D), jnp.float32),
        grid=(N // _BLK,),
        in_specs=[
            pl.BlockSpec((2, _BLK, D), lambda i: (0, i, 0)),
            pl.BlockSpec((1, _BLK, CNTW), lambda i: (rel, i, 0)),
            pl.BlockSpec((_BLK, D), lambda i: (i, 0)),
            pl.BlockSpec((2 * D, D), lambda i: (0, 0)),
            pl.BlockSpec((1, D), lambda i: (0, 0)),
        ],
        out_specs=pl.BlockSpec((_BLK, D), lambda i: (i, 0)),
    )(sums2, cnts, x, w, b.reshape(1, D))


def _pack_idx(flat, lead, per_tile, nchunk, pad_val):
    a = flat.reshape(lead, 16, per_tile)
    a = jnp.pad(a, ((0, 0), (0, 0), (0, nchunk * C - per_tile)),
                constant_values=pad_val)
    return a.reshape(lead, 16, nchunk, C)


def kernel(x_node, x_link, edge_index_nl, edge_index_ln,
           W_nl_0, b_nl_0, W_ln_0, b_ln_0,
           W_nl_1, b_nl_1, W_ln_1, b_ln_1):
    src_nl, dst_nl = edge_index_nl[0], edge_index_nl[1]
    src_ln, dst_ln = edge_index_ln[0], edge_index_ln[1]

    table0 = jnp.concatenate([x_node, x_link], axis=0)
    srcA = _pack_idx(jnp.concatenate([src_nl, src_ln + N]), 2, E // 16,
                     NCHUNK_A, 0)
    dstA = _pack_idx(jnp.concatenate([dst_nl, dst_ln]), 2, E // 16,
                     NCHUNK_A, N)
    srcC = _pack_idx(src_nl, 2, E // 32, NCHUNK_C, 0)
    dstC = _pack_idx(dst_nl, 2, E // 32, NCHUNK_C, N)

    ones16 = jnp.ones((C, CNTW), jnp.float32)
    zeros_f = jnp.zeros((N_ACC, D), jnp.float32)
    zeros_c = jnp.zeros((N_ACC, CNTW), jnp.float32)

    sums1, cnts = _sc_stage_a(table0, srcA, dstA, ones16, zeros_f, zeros_c)
    xl1 = _tc_update(sums1, cnts, 0, x_link, W_nl_0, b_nl_0)
    xn1 = _tc_update(sums1, cnts, 1, x_node, W_ln_0, b_ln_0)
    sums2 = _sc_stage_c(xn1, srcC, dstC, zeros_f)
    xl2 = _tc_update2(sums2, cnts, 0, xl1, W_nl_1, b_nl_1)
    return xl2


# speedup vs baseline: 3.5486x; 3.5486x over previous
import functools

import jax
import jax.numpy as jnp
from jax import lax
from jax.experimental import pallas as pl
from jax.experimental.pallas import tpu as pltpu
from jax.experimental.pallas import tpu_sc as plsc

N = 10000
D = 128
E = 320000
N_ACC = 10112
ROWS_PER_TILE = N_ACC // 16
C = 128
NCHUNK_A = 158
NCHUNK_C = 80
CNTW = 128


def _sc_sums_body(nchunk, table, src_idx, dst_idx, zeros_f, sums,
                  idx_s, idx_d, rows, acc, isem, gsem):
    c = lax.axis_index("c")
    s = lax.axis_index("s")
    r0 = s * ROWS_PER_TILE
    pltpu.sync_copy(zeros_f.at[pl.ds(r0, ROWS_PER_TILE)],
                    acc.at[pl.ds(r0, ROWS_PER_TILE)])
    plsc.subcore_barrier()

    def chunk(i, carry):
        pltpu.async_copy(src_idx.at[c, s, i], idx_s, isem)
        pltpu.async_copy(dst_idx.at[c, s, i], idx_d, isem)
        pltpu.make_async_copy(src_idx.at[c, s, i], idx_s, isem).wait()
        pltpu.make_async_copy(dst_idx.at[c, s, i], idx_d, isem).wait()
        pltpu.async_copy(table.at[idx_s], rows, gsem).wait()
        pltpu.sync_copy(rows, acc.at[idx_d], add=True)
        return carry

    lax.fori_loop(0, nchunk, chunk, 0)
    plsc.subcore_barrier()
    pltpu.sync_copy(acc.at[pl.ds(r0, ROWS_PER_TILE)],
                    sums.at[c, pl.ds(r0, ROWS_PER_TILE)])


def _sc_counts_body(dst_idx, ones_blk, zeros_c, cnts,
                    idx_d, ones_v, cacc, isem):
    c = lax.axis_index("c")
    s = lax.axis_index("s")
    r0 = s * ROWS_PER_TILE
    pltpu.sync_copy(ones_blk, ones_v)
    pltpu.sync_copy(zeros_c.at[pl.ds(r0, ROWS_PER_TILE)],
                    cacc.at[pl.ds(r0, ROWS_PER_TILE)])
    plsc.subcore_barrier()

    def chunk(i, carry):
        pltpu.async_copy(dst_idx.at[c, s, i], idx_d, isem).wait()
        pltpu.sync_copy(ones_v, cacc.at[idx_d], add=True)
        return carry

    lax.fori_loop(0, NCHUNK_A, chunk, 0)
    plsc.subcore_barrier()
    pltpu.sync_copy(cacc.at[pl.ds(r0, ROWS_PER_TILE)],
                    cnts.at[c, pl.ds(r0, ROWS_PER_TILE)])


@functools.lru_cache(maxsize=None)
def _sc_stages():
    mesh = plsc.VectorSubcoreMesh(core_axis_name="c", subcore_axis_name="s",
                                  num_cores=2, num_subcores=16)

    def sums_kernel(nchunk):
        return pl.kernel(
            functools.partial(_sc_sums_body, nchunk),
            out_type=jax.ShapeDtypeStruct((2, N_ACC, D), jnp.float32),
            mesh=mesh,
            scratch_types=[
                pltpu.VMEM((C,), jnp.int32),
                pltpu.VMEM((C,), jnp.int32),
                pltpu.VMEM((C, D), jnp.float32),
                pltpu.VMEM_SHARED((N_ACC, D), jnp.float32),
                pltpu.SemaphoreType.DMA,
                pltpu.SemaphoreType.DMA,
            ],
        )

    counts_kernel = pl.kernel(
        _sc_counts_body,
        out_type=jax.ShapeDtypeStruct((2, N_ACC, CNTW), jnp.float32),
        mesh=mesh,
        scratch_types=[
            pltpu.VMEM((C,), jnp.int32),
            pltpu.VMEM((C, CNTW), jnp.float32),
            pltpu.VMEM_SHARED((N_ACC, CNTW), jnp.float32),
            pltpu.SemaphoreType.DMA,
        ],
    )
    return sums_kernel(NCHUNK_A), sums_kernel(NCHUNK_C), counts_kernel


_BLK = 1000


def _tc_update_body(s_ref, c_ref, x_ref, w_ref, b_ref, o_ref):
    inv = 1.0 / jnp.maximum(c_ref[0, :, 0:1], 1.0)
    agg = s_ref[0] * inv
    xcat = jnp.concatenate([agg, x_ref[...]], axis=1)
    o_ref[...] = jnp.maximum(
        jnp.dot(xcat, w_ref[...], preferred_element_type=jnp.float32)
        + b_ref[...], 0.0)


def _tc_update(sums, cnts, rel, x, w, b):
    return pl.pallas_call(
        _tc_update_body,
        out_shape=jax.ShapeDtypeStruct((N, D), jnp.float32),
        grid=(N // _BLK,),
        in_specs=[
            pl.BlockSpec((1, _BLK, D), lambda i: (rel, i, 0)),
            pl.BlockSpec((1, _BLK, CNTW), lambda i: (rel, i, 0)),
            pl.BlockSpec((_BLK, D), lambda i: (i, 0)),
            pl.BlockSpec((2 * D, D), lambda i: (0, 0)),
            pl.BlockSpec((1, D), lambda i: (0, 0)),
        ],
        out_specs=pl.BlockSpec((_BLK, D), lambda i: (i, 0)),
    )(sums, cnts, x, w, b.reshape(1, D))


def _tc_update2_body(s_ref, c_ref, x_ref, w_ref, b_ref, o_ref):
    inv = 1.0 / jnp.maximum(c_ref[0, :, 0:1], 1.0)
    agg = (s_ref[0] + s_ref[1]) * inv
    xcat = jnp.concatenate([agg, x_ref[...]], axis=1)
    o_ref[...] = jnp.maximum(
        jnp.dot(xcat, w_ref[...], preferred_element_type=jnp.float32)
        + b_ref[...], 0.0)


def _tc_update2(sums2, cnts, rel, x, w, b):
    return pl.pallas_call(
        _tc_update2_body,
        out_shape=jax.ShapeDtypeStruct((N, D), jnp.float32),
        grid=(N // _BLK,),
        in_specs=[
            pl.BlockSpec((2, _BLK, D), lambda i: (0, i, 0)),
            pl.BlockSpec((1, _BLK, CNTW), lambda i: (rel, i, 0)),
            pl.BlockSpec((_BLK, D), lambda i: (i, 0)),
            pl.BlockSpec((2 * D, D), lambda i: (0, 0)),
            pl.BlockSpec((1, D), lambda i: (0, 0)),
        ],
        out_specs=pl.BlockSpec((_BLK, D), lambda i: (i, 0)),
    )(sums2, cnts, x, w, b.reshape(1, D))


def _pack_idx(flat, lead, per_tile, nchunk, pad_val):
    a = flat.reshape(lead, 16, per_tile)
    a = jnp.pad(a, ((0, 0), (0, 0), (0, nchunk * C - per_tile)),
                constant_values=pad_val)
    return a.reshape(lead, 16, nchunk, C)


def kernel(x_node, x_link, edge_index_nl, edge_index_ln,
           W_nl_0, b_nl_0, W_ln_0, b_ln_0,
           W_nl_1, b_nl_1, W_ln_1, b_ln_1):
    src_nl, dst_nl = edge_index_nl[0], edge_index_nl[1]
    src_ln, dst_ln = edge_index_ln[0], edge_index_ln[1]

    table0 = jnp.concatenate([x_node, x_link], axis=0)
    srcA = _pack_idx(jnp.concatenate([src_nl, src_ln + N]), 2, E // 16,
                     NCHUNK_A, 0)
    dstA = _pack_idx(jnp.concatenate([dst_nl, dst_ln]), 2, E // 16,
                     NCHUNK_A, N)
    srcC = _pack_idx(src_nl, 2, E // 32, NCHUNK_C, 0)
    dstC = _pack_idx(dst_nl, 2, E // 32, NCHUNK_C, N)

    ones_blk = jnp.ones((C, CNTW), jnp.float32)
    zeros_f = jnp.zeros((N_ACC, D), jnp.float32)
    zeros_c = jnp.zeros((N_ACC, CNTW), jnp.float32)

    sums_a, sums_c, counts = _sc_stages()
    cnts = counts(dstA, ones_blk, zeros_c)
    sums1 = sums_a(table0, srcA, dstA, zeros_f)
    xl1 = _tc_update(sums1, cnts, 0, x_link, W_nl_0, b_nl_0)
    xn1 = _tc_update(sums1, cnts, 1, x_node, W_ln_0, b_ln_0)
    sums2 = sums_c(xn1, srcC, dstC, zeros_f)
    xl2 = _tc_update2(sums2, cnts, 0, xl1, W_nl_1, b_nl_1)
    return xl2
